# Initial kernel scaffold; baseline (speedup 1.0000x reference)
import jax
import jax.numpy as jnp
from jax.experimental import pallas as pl


def kernel(inputs, pos_table):
    raise NotImplementedError("write your pallas kernel here")


# speedup vs baseline: 1.4580x; 1.4580x over previous
import jax
import jax.numpy as jnp
from jax.experimental import pallas as pl


_BS = 512


def _add_kernel(x_ref, p_ref, o_ref):
    o_ref[...] = x_ref[...] + p_ref[...]


def kernel(inputs, pos_table):
    b, s, d = inputs.shape
    grid = (b, s // _BS)
    return pl.pallas_call(
        _add_kernel,
        grid=grid,
        in_specs=[
            pl.BlockSpec((1, _BS, d), lambda i, j: (i, j, 0)),
            pl.BlockSpec((_BS, d), lambda i, j: (j, 0)),
        ],
        out_specs=pl.BlockSpec((1, _BS, d), lambda i, j: (i, j, 0)),
        out_shape=jax.ShapeDtypeStruct((b, s, d), inputs.dtype),
    )(inputs, pos_table)


# speedup vs baseline: 1.6720x; 1.1468x over previous
import jax
import jax.numpy as jnp
from jax.experimental import pallas as pl


_BS = 512


def _add_kernel(x_ref, p_ref, o_ref):
    o_ref[...] = x_ref[...] + p_ref[...]


def kernel(inputs, pos_table):
    b, s, d = inputs.shape
    grid = (s // _BS, b)
    return pl.pallas_call(
        _add_kernel,
        grid=grid,
        in_specs=[
            pl.BlockSpec((1, _BS, d), lambda j, i: (i, j, 0)),
            pl.BlockSpec((_BS, d), lambda j, i: (j, 0)),
        ],
        out_specs=pl.BlockSpec((1, _BS, d), lambda j, i: (i, j, 0)),
        out_shape=jax.ShapeDtypeStruct((b, s, d), inputs.dtype),
    )(inputs, pos_table)


# speedup vs baseline: 1.8551x; 1.1095x over previous
import jax
import jax.numpy as jnp
from jax.experimental import pallas as pl


_BS = 1024


def _add_kernel(x_ref, p_ref, o_ref):
    o_ref[...] = x_ref[...] + p_ref[...]


def kernel(inputs, pos_table):
    b, s, d = inputs.shape
    grid = (s // _BS, b)
    return pl.pallas_call(
        _add_kernel,
        grid=grid,
        in_specs=[
            pl.BlockSpec((1, _BS, d), lambda j, i: (i, j, 0)),
            pl.BlockSpec((_BS, d), lambda j, i: (j, 0)),
        ],
        out_specs=pl.BlockSpec((1, _BS, d), lambda j, i: (i, j, 0)),
        out_shape=jax.ShapeDtypeStruct((b, s, d), inputs.dtype),
    )(inputs, pos_table)


# speedup vs baseline: 1.9725x; 1.0633x over previous
import jax
import jax.numpy as jnp
from jax.experimental import pallas as pl


_BS = 2048


def _add_kernel(x_ref, p_ref, o_ref):
    o_ref[...] = x_ref[...] + p_ref[...]


def kernel(inputs, pos_table):
    b, s, d = inputs.shape
    grid = (s // _BS, b)
    return pl.pallas_call(
        _add_kernel,
        grid=grid,
        in_specs=[
            pl.BlockSpec((1, _BS, d), lambda j, i: (i, j, 0)),
            pl.BlockSpec((_BS, d), lambda j, i: (j, 0)),
        ],
        out_specs=pl.BlockSpec((1, _BS, d), lambda j, i: (i, j, 0)),
        out_shape=jax.ShapeDtypeStruct((b, s, d), inputs.dtype),
    )(inputs, pos_table)
